# Initial kernel scaffold; baseline (speedup 1.0000x reference)
import jax
import jax.numpy as jnp
from jax.experimental import pallas as pl


def kernel(adj, feature, w1, b1, w2, b2):
    raise NotImplementedError("write your pallas kernel here")


# speedup vs baseline: 3.7031x; 3.7031x over previous
import jax
import jax.numpy as jnp
from jax import lax
from jax.experimental import pallas as pl
from jax.experimental.pallas import tpu as pltpu


def _p1_kernel(a_ref, x_ref, wt_ref, yt_ref, din_ref, doutp_ref, a8_ref):
    k = pl.program_id(1)
    a = a_ref[...]
    x = x_ref[...]

    yt_ref[...] = lax.dot_general(
        wt_ref[...], x, (((1,), (1,)), ((), ())),
        preferred_element_type=jnp.float32)

    ones_row = jnp.ones((1, a.shape[1]), jnp.float32)
    din_ref[...] = lax.dot_general(
        ones_row, a, (((1,), (1,)), ((), ())),
        preferred_element_type=jnp.float32)

    @pl.when(k == 0)
    def _():
        doutp_ref[...] = jnp.zeros_like(doutp_ref)

    ones_col = jnp.ones((1, a.shape[0]), jnp.float32)
    doutp_ref[0] += lax.dot_general(
        ones_col, a, (((1,), (0,)), ((), ())),
        preferred_element_type=jnp.float32)

    a8_ref[...] = a.astype(jnp.int8)


def _p2_kernel(a8_ref, ytk_ref, doutp_ref, ytm_ref, din_ref, b1_ref, b2_ref,
               score_ref, acc_ref):
    k = pl.program_id(1)

    @pl.when(k == 0)
    def _():
        acc_ref[...] = jnp.zeros_like(acc_ref)

    a = a8_ref[...].astype(jnp.float32)
    dout = doutp_ref[0] + doutp_ref[1]
    inv_out = lax.rsqrt(jnp.maximum(dout, 1.0))
    z = inv_out * ytk_ref[0:1, :]
    acc_ref[...] += lax.dot_general(
        z, a, (((1,), (1,)), ((), ())),
        preferred_element_type=jnp.float32)

    @pl.when(k == pl.num_programs(1) - 1)
    def _():
        inv_in = lax.rsqrt(jnp.maximum(din_ref[...], 1.0))
        s1 = inv_in * acc_ref[...] + b1_ref[0, 0]
        s2 = ytm_ref[1:2, :] + b2_ref[0, 0]
        score_ref[...] = jnp.maximum(s1, s2)


def _scale_kernel(x_ref, s_ref, o_ref):
    o_ref[...] = x_ref[...] * jnp.tanh(s_ref[...])


def kernel(adj, feature, w1, b1, w2, b2):
    n, d = feature.shape
    n_graphs = 8
    seg = n // n_graphs
    kk = -(-seg // 2)

    wt = jnp.concatenate([w1, w2], axis=1).T.astype(jnp.float32)

    nbk = 8
    rb = n // (2 * nbk)
    yt, din, doutp, a8 = pl.pallas_call(
        _p1_kernel,
        out_shape=(
            jax.ShapeDtypeStruct((2, n), jnp.float32),
            jax.ShapeDtypeStruct((1, n), jnp.float32),
            jax.ShapeDtypeStruct((2, 1, n), jnp.float32),
            jax.ShapeDtypeStruct((n, n), jnp.int8),
        ),
        grid=(2, nbk),
        in_specs=[
            pl.BlockSpec((rb, n), lambda i, k: (i * nbk + k, 0)),
            pl.BlockSpec((rb, d), lambda i, k: (i * nbk + k, 0)),
            pl.BlockSpec((2, d), lambda i, k: (0, 0)),
        ],
        out_specs=(
            pl.BlockSpec((2, rb), lambda i, k: (0, i * nbk + k)),
            pl.BlockSpec((1, rb), lambda i, k: (0, i * nbk + k)),
            pl.BlockSpec((1, 1, n), lambda i, k: (i, 0, 0)),
            pl.BlockSpec((rb, n), lambda i, k: (i * nbk + k, 0)),
        ),
        compiler_params=pltpu.CompilerParams(
            dimension_semantics=("parallel", "arbitrary")),
    )(adj, feature, wt)

    m = n // 2
    tk = min(1024, n)
    score_row = pl.pallas_call(
        _p2_kernel,
        out_shape=jax.ShapeDtypeStruct((1, n), jnp.float32),
        grid=(2, n // tk),
        in_specs=[
            pl.BlockSpec((m, tk), lambda i, k: (i, k)),
            pl.BlockSpec((2, tk), lambda i, k: (0, k)),
            pl.BlockSpec((2, 1, tk), lambda i, k: (0, 0, k)),
            pl.BlockSpec((2, m), lambda i, k: (0, i)),
            pl.BlockSpec((1, m), lambda i, k: (0, i)),
            pl.BlockSpec(memory_space=pltpu.MemorySpace.SMEM),
            pl.BlockSpec(memory_space=pltpu.MemorySpace.SMEM),
        ],
        out_specs=pl.BlockSpec((1, m), lambda i, k: (0, i)),
        scratch_shapes=[pltpu.VMEM((1, m), jnp.float32)],
        compiler_params=pltpu.CompilerParams(
            dimension_semantics=("parallel", "arbitrary")),
    )(a8, yt, doutp, yt, din, b1, b2)
    score = score_row[0]

    _, idx = lax.top_k(score.reshape(n_graphs, seg), kk)
    offs = (seg * jnp.arange(n_graphs, dtype=idx.dtype))[:, None]
    perm = (idx + offs).reshape(-1)

    big_k = perm.shape[0]
    xp = feature[perm]
    sp = score[perm].reshape(big_k, 1)
    pooled = pl.pallas_call(
        _scale_kernel,
        out_shape=jax.ShapeDtypeStruct((big_k, d), jnp.float32),
        grid=(n_graphs,),
        in_specs=[
            pl.BlockSpec((big_k // n_graphs, d), lambda i: (i, 0)),
            pl.BlockSpec((big_k // n_graphs, 1), lambda i: (i, 0)),
        ],
        out_specs=pl.BlockSpec((big_k // n_graphs, d), lambda i: (i, 0)),
        compiler_params=pltpu.CompilerParams(
            dimension_semantics=("parallel",)),
    )(xp, sp)

    sub_adj = adj[perm[:, None], perm[None, :]]
    return sub_adj, pooled, perm, [kk] * n_graphs


# speedup vs baseline: 5.9262x; 1.6003x over previous
import functools

import jax
import jax.numpy as jnp
from jax import lax
from jax.experimental import pallas as pl
from jax.experimental.pallas import tpu as pltpu


def _p1_kernel(a_ref, x_ref, wt_ref, yt_ref, din_ref, doutp_ref, a8_ref):
    k = pl.program_id(1)
    a = a_ref[...]
    x = x_ref[...]

    yt_ref[...] = lax.dot_general(
        wt_ref[...], x, (((1,), (1,)), ((), ())),
        preferred_element_type=jnp.float32)

    ones_row = jnp.ones((1, a.shape[1]), jnp.float32)
    din_ref[...] = lax.dot_general(
        ones_row, a, (((1,), (1,)), ((), ())),
        preferred_element_type=jnp.float32)

    @pl.when(k == 0)
    def _():
        doutp_ref[...] = jnp.zeros_like(doutp_ref)

    ones_col = jnp.ones((1, a.shape[0]), jnp.float32)
    doutp_ref[0] += lax.dot_general(
        ones_col, a, (((1,), (0,)), ((), ())),
        preferred_element_type=jnp.float32)

    a8_ref[...] = a.astype(jnp.int8)


def _p2_kernel(a8_ref, ytk_ref, doutp_ref, ytm_ref, din_ref, b1_ref, b2_ref,
               score_ref, acc_ref):
    k = pl.program_id(1)

    @pl.when(k == 0)
    def _():
        acc_ref[...] = jnp.zeros_like(acc_ref)

    a = a8_ref[...].astype(jnp.float32)
    dout = doutp_ref[0] + doutp_ref[1]
    inv_out = lax.rsqrt(jnp.maximum(dout, 1.0))
    z = inv_out * ytk_ref[0:1, :]
    acc_ref[...] += lax.dot_general(
        z, a, (((1,), (1,)), ((), ())),
        preferred_element_type=jnp.float32)

    @pl.when(k == pl.num_programs(1) - 1)
    def _():
        inv_in = lax.rsqrt(jnp.maximum(din_ref[...], 1.0))
        s1 = inv_in * acc_ref[...] + b1_ref[0, 0]
        s2 = ytm_ref[1:2, :] + b2_ref[0, 0]
        score_ref[...] = jnp.maximum(s1, s2)


def _scale_kernel(x_ref, s_ref, o_ref):
    o_ref[...] = x_ref[...] * jnp.tanh(s_ref[...])


def _colsel_kernel(r8_ref, perm_ref, o_ref, *, seg, kk, n_graphs):
    r = r8_ref[...].astype(jnp.bfloat16)
    for g in range(n_graphs):
        pg = perm_ref[0:1, g * kk:(g + 1) * kk] - g * seg
        iota = lax.broadcasted_iota(jnp.int32, (seg, kk), 0)
        onehot = (iota == pg).astype(jnp.bfloat16)
        o_ref[:, g * kk:(g + 1) * kk] = lax.dot_general(
            r[:, g * seg:(g + 1) * seg], onehot,
            (((1,), (0,)), ((), ())),
            preferred_element_type=jnp.float32)


def kernel(adj, feature, w1, b1, w2, b2):
    n, d = feature.shape
    n_graphs = 8
    seg = n // n_graphs
    kk = -(-seg // 2)

    wt = jnp.concatenate([w1, w2], axis=1).T.astype(jnp.float32)

    nbk = 8
    rb = n // (2 * nbk)
    yt, din, doutp, a8 = pl.pallas_call(
        _p1_kernel,
        out_shape=(
            jax.ShapeDtypeStruct((2, n), jnp.float32),
            jax.ShapeDtypeStruct((1, n), jnp.float32),
            jax.ShapeDtypeStruct((2, 1, n), jnp.float32),
            jax.ShapeDtypeStruct((n, n), jnp.int8),
        ),
        grid=(2, nbk),
        in_specs=[
            pl.BlockSpec((rb, n), lambda i, k: (i * nbk + k, 0)),
            pl.BlockSpec((rb, d), lambda i, k: (i * nbk + k, 0)),
            pl.BlockSpec((2, d), lambda i, k: (0, 0)),
        ],
        out_specs=(
            pl.BlockSpec((2, rb), lambda i, k: (0, i * nbk + k)),
            pl.BlockSpec((1, rb), lambda i, k: (0, i * nbk + k)),
            pl.BlockSpec((1, 1, n), lambda i, k: (i, 0, 0)),
            pl.BlockSpec((rb, n), lambda i, k: (i * nbk + k, 0)),
        ),
        compiler_params=pltpu.CompilerParams(
            dimension_semantics=("parallel", "arbitrary")),
    )(adj, feature, wt)

    m = n // 2
    tk = min(1024, n)
    score_row = pl.pallas_call(
        _p2_kernel,
        out_shape=jax.ShapeDtypeStruct((1, n), jnp.float32),
        grid=(2, n // tk),
        in_specs=[
            pl.BlockSpec((m, tk), lambda i, k: (i, k)),
            pl.BlockSpec((2, tk), lambda i, k: (0, k)),
            pl.BlockSpec((2, 1, tk), lambda i, k: (0, 0, k)),
            pl.BlockSpec((2, m), lambda i, k: (0, i)),
            pl.BlockSpec((1, m), lambda i, k: (0, i)),
            pl.BlockSpec(memory_space=pltpu.MemorySpace.SMEM),
            pl.BlockSpec(memory_space=pltpu.MemorySpace.SMEM),
        ],
        out_specs=pl.BlockSpec((1, m), lambda i, k: (0, i)),
        scratch_shapes=[pltpu.VMEM((1, m), jnp.float32)],
        compiler_params=pltpu.CompilerParams(
            dimension_semantics=("parallel", "arbitrary")),
    )(a8, yt, doutp, yt, din, b1, b2)
    score = score_row[0]

    _, idx = lax.top_k(score.reshape(n_graphs, seg), kk)
    offs = (seg * jnp.arange(n_graphs, dtype=idx.dtype))[:, None]
    perm = (idx + offs).reshape(-1)

    big_k = perm.shape[0]
    xp = feature[perm]
    sp = score[perm].reshape(big_k, 1)
    pooled = pl.pallas_call(
        _scale_kernel,
        out_shape=jax.ShapeDtypeStruct((big_k, d), jnp.float32),
        grid=(n_graphs,),
        in_specs=[
            pl.BlockSpec((big_k // n_graphs, d), lambda i: (i, 0)),
            pl.BlockSpec((big_k // n_graphs, 1), lambda i: (i, 0)),
        ],
        out_specs=pl.BlockSpec((big_k // n_graphs, d), lambda i: (i, 0)),
        compiler_params=pltpu.CompilerParams(
            dimension_semantics=("parallel",)),
    )(xp, sp)

    r8 = a8[perm]
    rbo = big_k // 4
    sub_adj = pl.pallas_call(
        functools.partial(_colsel_kernel, seg=seg, kk=kk, n_graphs=n_graphs),
        out_shape=jax.ShapeDtypeStruct((big_k, big_k), jnp.float32),
        grid=(2, 2),
        in_specs=[
            pl.BlockSpec((rbo, n), lambda i, k: (i * 2 + k, 0)),
            pl.BlockSpec((1, big_k), lambda i, k: (0, 0)),
        ],
        out_specs=pl.BlockSpec((rbo, big_k), lambda i, k: (i * 2 + k, 0)),
        compiler_params=pltpu.CompilerParams(
            dimension_semantics=("parallel", "arbitrary")),
    )(r8, perm.reshape(1, big_k))
    return sub_adj, pooled, perm, [kk] * n_graphs
